# baseline (speedup 1.0000x reference)
import functools

import jax
import jax.numpy as jnp
from jax import lax
from jax.experimental import pallas as pl
from jax.experimental.pallas import tpu as pltpu
from jax.experimental.pallas import tpu_sc as plsc

_MAX_SPIKE = 100.0
_P = 0.1
_NC, _NS = 2, 16
_NW = _NC * _NS
_CW = 2048


def _spike_consts(B, T, dtype):
    key = jax.random.key(42)
    k1, k2, k3 = jax.random.split(key, 3)
    probas = jax.random.uniform(k1, (B,), dtype=jnp.float32)
    mask = probas > (1.0 - _P)
    pos = jax.random.randint(k2, (B,), 0, T - 2)
    sign = jnp.where(jax.random.randint(k3, (), 0, 2) == 0, -1.0, 1.0).astype(dtype)
    vals = jnp.where(mask, sign * _MAX_SPIKE, 0.0).astype(dtype)
    return pos, vals


def _sc_body(B, T, x_hbm, pos_hbm, val_hbm, out_hbm, posv, valv, win,
             sem_bulk, sem_win):
    rows = B // _NW
    ngrp = rows // 16
    wid = lax.axis_index("s") * _NC + lax.axis_index("c")
    r0 = wid * rows
    bulk = pltpu.async_copy(x_hbm.at[pl.ds(r0, rows)],
                            out_hbm.at[pl.ds(r0, rows)], sem_bulk)
    pltpu.sync_copy(pos_hbm.at[pl.ds(r0, rows)], posv)
    pltpu.sync_copy(val_hbm.at[pl.ds(r0, rows)], valv)
    lane = lax.broadcasted_iota(jnp.int32, (16,), 0)

    w0s, offs, vs = [], [], []
    gather_h = []
    for j in range(rows):
        g, l = divmod(j, 16)
        pos16 = posv[pl.ds(g * 16, 16)]
        val16 = valv[pl.ds(g * 16, 16)]
        p = pos16[l]
        w0 = (p // 16) * 16
        w0s.append(w0)
        offs.append(p - w0)
        vs.append(val16[l])
        gather_h.append(pltpu.async_copy(
            x_hbm.at[r0 + j, pl.ds(w0, 16)], win.at[j], sem_win))
    for j in range(rows):
        gather_h[j].wait()
        win[j] = win[j] + jnp.where(lane == offs[j], vs[j], 0.0)
    bulk.wait()
    scatter_h = [pltpu.async_copy(win.at[j],
                                  out_hbm.at[r0 + j, pl.ds(w0s[j], 16)],
                                  sem_win)
                 for j in range(rows)]
    for h in scatter_h:
        h.wait()


def kernel(x):
    B, T = x.shape
    pos, vals = _spike_consts(B, T, x.dtype)
    mesh = plsc.VectorSubcoreMesh(core_axis_name="c", subcore_axis_name="s",
                                  num_cores=_NC, num_subcores=_NS)
    rows = B // _NW
    sc_call = pl.kernel(
        functools.partial(_sc_body, B, T),
        out_type=jax.ShapeDtypeStruct((B, T), x.dtype),
        mesh=mesh,
        compiler_params=pltpu.CompilerParams(needs_layout_passes=False),
        scratch_types=[
            pltpu.VMEM((rows,), jnp.int32),
            pltpu.VMEM((rows,), jnp.float32),
            pltpu.VMEM((rows, 16), jnp.float32),
            pltpu.SemaphoreType.DMA,
            pltpu.SemaphoreType.DMA,
        ],
    )
    return sc_call(x, pos, vals)


# speedup vs baseline: 29.4386x; 29.4386x over previous
import functools

import jax
import jax.numpy as jnp
from jax import lax
from jax.experimental import pallas as pl
from jax.experimental.pallas import tpu as pltpu
from jax.experimental.pallas import tpu_sc as plsc

_MAX_SPIKE = 100.0
_P = 0.1
_NC, _NS = 2, 16
_NW = _NC * _NS
_CW = 4096


def _spike_consts(B, T, dtype):
    key = jax.random.key(42)
    k1, k2, k3 = jax.random.split(key, 3)
    probas = jax.random.uniform(k1, (B,), dtype=jnp.float32)
    mask = probas > (1.0 - _P)
    pos = jax.random.randint(k2, (B,), 0, T - 2)
    sign = jnp.where(jax.random.randint(k3, (), 0, 2) == 0, -1.0, 1.0).astype(dtype)
    vals = jnp.where(mask, sign * _MAX_SPIKE, 0.0).astype(dtype)
    return pos, vals


def _sc_body(B, T, x_hbm, pos_hbm, val_hbm, out_hbm, posv, valv, buf,
             sem_in, sem_out):
    rows = B // _NW
    nband = rows // 8
    nch = T // _CW
    nslab = nband * nch
    wid = lax.axis_index("s") * _NC + lax.axis_index("c")
    r0 = wid * rows
    pltpu.sync_copy(pos_hbm.at[pl.ds(r0, rows)], posv)
    pltpu.sync_copy(val_hbm.at[pl.ds(r0, rows)], valv)
    lane = lax.broadcasted_iota(jnp.int32, (16,), 0)
    pos16 = [posv[pl.ds(g * 16, 16)] for g in range(rows // 16)]
    val16 = [valv[pl.ds(g * 16, 16)] for g in range(rows // 16)]

    def slab_src(j):
        b, c = divmod(j, nch)
        return x_hbm.at[pl.ds(r0 + b * 8, 8), pl.ds(c * _CW, _CW)]

    def slab_dst(j):
        b, c = divmod(j, nch)
        return out_hbm.at[pl.ds(r0 + b * 8, 8), pl.ds(c * _CW, _CW)]

    def spike(j):
        b, c = divmod(j, nch)
        g, h = divmod(b, 2)
        base = c * _CW
        inside = ((lane >> 3) == h) & (pos16[g] >= base) \
            & (pos16[g] < base + _CW)
        col = jnp.clip(pos16[g] - base, 0, _CW - 1)
        plsc.addupdate_scatter(buf.at[j % 2], [lane & 7, col], val16[g],
                               mask=inside)

    in_h = [None] * nslab
    out_h = [None] * nslab
    in_h[0] = pltpu.async_copy(slab_src(0), buf.at[0], sem_in)
    for j in range(nslab):
        if j + 1 < nslab:
            if j - 1 >= 0:
                out_h[j - 1].wait()
            in_h[j + 1] = pltpu.async_copy(
                slab_src(j + 1), buf.at[(j + 1) % 2], sem_in)
        in_h[j].wait()
        spike(j)
        out_h[j] = pltpu.async_copy(buf.at[j % 2], slab_dst(j), sem_out)
    out_h[nslab - 2].wait()
    out_h[nslab - 1].wait()


def kernel(x):
    B, T = x.shape
    pos, vals = _spike_consts(B, T, x.dtype)
    mesh = plsc.VectorSubcoreMesh(core_axis_name="c", subcore_axis_name="s",
                                  num_cores=_NC, num_subcores=_NS)
    rows = B // _NW
    sc_call = pl.kernel(
        functools.partial(_sc_body, B, T),
        out_type=jax.ShapeDtypeStruct((B, T), x.dtype),
        mesh=mesh,
        compiler_params=pltpu.CompilerParams(needs_layout_passes=False),
        scratch_types=[
            pltpu.VMEM((rows,), jnp.int32),
            pltpu.VMEM((rows,), jnp.float32),
            pltpu.VMEM((2, 8, _CW), jnp.float32),
            pltpu.SemaphoreType.DMA,
            pltpu.SemaphoreType.DMA,
        ],
    )
    return sc_call(x, pos, vals)


# speedup vs baseline: 29.6065x; 1.0057x over previous
import functools

import jax
import jax.numpy as jnp
from jax import lax
from jax.experimental import pallas as pl
from jax.experimental.pallas import tpu as pltpu
from jax.experimental.pallas import tpu_sc as plsc

_MAX_SPIKE = 100.0
_P = 0.1
_NC, _NS = 2, 16
_NW = _NC * _NS
_CW = 4096
_DEPTH = 3


def _spike_consts(B, T, dtype):
    key = jax.random.key(42)
    k1, k2, k3 = jax.random.split(key, 3)
    probas = jax.random.uniform(k1, (B,), dtype=jnp.float32)
    mask = probas > (1.0 - _P)
    pos = jax.random.randint(k2, (B,), 0, T - 2)
    sign = jnp.where(jax.random.randint(k3, (), 0, 2) == 0, -1.0, 1.0).astype(dtype)
    vals = jnp.where(mask, sign * _MAX_SPIKE, 0.0).astype(dtype)
    return pos, vals


def _sc_body(B, T, x_hbm, pos_hbm, val_hbm, out_hbm, posv, valv, buf,
             sem_in, sem_out):
    rows = B // _NW
    nband = rows // 8
    nch = T // _CW
    nslab = nband * nch
    wid = lax.axis_index("s") * _NC + lax.axis_index("c")
    r0 = wid * rows
    pltpu.sync_copy(pos_hbm.at[pl.ds(r0, rows)], posv)
    pltpu.sync_copy(val_hbm.at[pl.ds(r0, rows)], valv)
    lane = lax.broadcasted_iota(jnp.int32, (16,), 0)
    pos16 = [posv[pl.ds(g * 16, 16)] for g in range(rows // 16)]
    val16 = [valv[pl.ds(g * 16, 16)] for g in range(rows // 16)]

    def slab_src(j):
        b, c = divmod(j, nch)
        return x_hbm.at[pl.ds(r0 + b * 8, 8), pl.ds(c * _CW, _CW)]

    def slab_dst(j):
        b, c = divmod(j, nch)
        return out_hbm.at[pl.ds(r0 + b * 8, 8), pl.ds(c * _CW, _CW)]

    def spike(j):
        b, c = divmod(j, nch)
        g, h = divmod(b, 2)
        base = c * _CW
        inside = ((lane >> 3) == h) & (pos16[g] >= base) \
            & (pos16[g] < base + _CW)
        col = jnp.clip(pos16[g] - base, 0, _CW - 1)
        plsc.addupdate_scatter(buf.at[j % _DEPTH], [lane & 7, col], val16[g],
                               mask=inside)

    in_h = [None] * nslab
    out_h = [None] * nslab
    for k in range(_DEPTH - 1):
        in_h[k] = pltpu.async_copy(slab_src(k), buf.at[k % _DEPTH], sem_in)
    for j in range(nslab):
        nxt = j + _DEPTH - 1
        if nxt < nslab:
            if nxt - _DEPTH >= 0:
                out_h[nxt - _DEPTH].wait()
            in_h[nxt] = pltpu.async_copy(
                slab_src(nxt), buf.at[nxt % _DEPTH], sem_in)
        in_h[j].wait()
        spike(j)
        out_h[j] = pltpu.async_copy(buf.at[j % _DEPTH], slab_dst(j), sem_out)
    for j in range(max(0, nslab - _DEPTH), nslab):
        out_h[j].wait()


def kernel(x):
    B, T = x.shape
    pos, vals = _spike_consts(B, T, x.dtype)
    mesh = plsc.VectorSubcoreMesh(core_axis_name="c", subcore_axis_name="s",
                                  num_cores=_NC, num_subcores=_NS)
    rows = B // _NW
    sc_call = pl.kernel(
        functools.partial(_sc_body, B, T),
        out_type=jax.ShapeDtypeStruct((B, T), x.dtype),
        mesh=mesh,
        compiler_params=pltpu.CompilerParams(needs_layout_passes=False),
        scratch_types=[
            pltpu.VMEM((rows,), jnp.int32),
            pltpu.VMEM((rows,), jnp.float32),
            pltpu.VMEM((_DEPTH, 8, _CW), jnp.float32),
            pltpu.SemaphoreType.DMA,
            pltpu.SemaphoreType.DMA,
        ],
    )
    return sc_call(x, pos, vals)
